# baseline (speedup 1.0000x reference)
import functools

import jax
import jax.numpy as jnp
from jax import lax
from jax.experimental import pallas as pl
from jax.experimental.pallas import tpu as pltpu
from jax.experimental.pallas import tpu_sc as plsc

_EPS = 1e-09


def _log_sc(x):
    xi = lax.bitcast_convert_type(x, jnp.int32)
    t = (xi.astype(jnp.float32) - 1064866805.0) * 8.262958405176314e-08
    for _ in range(3):
        t = t + x * jnp.exp(-t) - 1.0
    return t


def _mlp_body(x_ref, y_ref, w1_ref, b1_ref, w2_ref, b2_ref, w3_ref, b3_ref,
              es_ref, ets_ref):
    xh = x_ref[...].astype(jnp.bfloat16)
    w1h = (w1_ref[...] * 0.5).astype(jnp.bfloat16)
    a = jnp.dot(xh, w1h, preferred_element_type=jnp.float32) \
        + 0.5 * b1_ref[...]
    h = a + a * jnp.tanh(a)
    w2h = (w2_ref[...] * 0.5).astype(jnp.bfloat16)
    a = jnp.dot(h.astype(jnp.bfloat16), w2h,
                preferred_element_type=jnp.float32) + 0.5 * b2_ref[...]
    h = a + a * jnp.tanh(a)
    s = lax.dot_general(
        w3_ref[...].astype(jnp.bfloat16), h.astype(jnp.bfloat16),
        dimension_numbers=(((1,), (1,)), ((), ())),
        preferred_element_type=jnp.float32,
    ) + b3_ref[0]
    es_ref[...] = jnp.exp(s)[None]
    ets_ref[...] = (jnp.exp(y_ref[0]) * s)[None]


def _run_mlp(x, y3, W1, b1, W2, b2, W3, b3, block_m):
    n, d = x.shape
    h = W1.shape[1]
    grid = (n // block_m,)
    nb = n // block_m
    vec = pl.BlockSpec((1, 1, block_m), lambda i: (i, 0, 0))
    out3 = jax.ShapeDtypeStruct((nb, 1, block_m), jnp.float32)
    return pl.pallas_call(
        _mlp_body,
        grid=grid,
        in_specs=[
            pl.BlockSpec((block_m, d), lambda i: (i, 0)),
            vec,
            pl.BlockSpec((d, h), lambda i: (0, 0)),
            pl.BlockSpec((1, h), lambda i: (0, 0)),
            pl.BlockSpec((h, h), lambda i: (0, 0)),
            pl.BlockSpec((1, h), lambda i: (0, 0)),
            pl.BlockSpec((1, h), lambda i: (0, 0)),
            pl.BlockSpec(memory_space=pltpu.SMEM),
        ],
        out_specs=(vec, vec),
        out_shape=(out3, out3),
    )(x, y3, W1, b1.reshape(1, h), W2, b2.reshape(1, h), W3.reshape(1, h), b3)


def _sc_dent_body(y_hbm, g_hbm, dtp_hbm,
                  y_v, g_v, zero_v, dent_sh):
    cid = lax.axis_index("c")
    sid = lax.axis_index("s")
    row0 = cid * 256 + sid * 16

    pltpu.sync_copy(y_hbm.at[pl.ds(row0, 16)], y_v)
    pltpu.sync_copy(g_hbm.at[pl.ds(row0, 16)], g_v)

    def _expy_row(r, _):
        for j in range(8):
            c = pl.ds(16 * j, 16)
            y_v[r, c] = jnp.exp(y_v[r, c])
        return 0
    lax.fori_loop(0, 16, _expy_row, 0)

    @pl.when(sid == 0)
    def _zero():
        def _z(i, _):
            zero_v[pl.ds(16 * i, 16)] = jnp.zeros((16,), jnp.float32)
            return 0
        lax.fori_loop(0, 128, _z, 0)
        pltpu.sync_copy(zero_v, dent_sh)

    plsc.subcore_barrier()

    def _scat(j, _):
        pltpu.sync_copy(y_v.at[j], dent_sh.at[g_v.at[j]], add=True)
        return 0
    lax.fori_loop(0, 16, _scat, 0)

    plsc.subcore_barrier()

    @pl.when(sid == 0)
    def _out():
        pltpu.sync_copy(dent_sh, dtp_hbm.at[cid])


def _run_sc_dent(y2, g2, num_groups):
    mesh = plsc.VectorSubcoreMesh(core_axis_name="c", subcore_axis_name="s")
    f32 = jnp.float32
    kern = pl.kernel(
        _sc_dent_body,
        compiler_params=pltpu.CompilerParams(needs_layout_passes=False),
        out_type=jax.ShapeDtypeStruct((2, num_groups), f32),
        mesh=mesh,
        scratch_types=[
            pltpu.VMEM((16, 128), f32),
            pltpu.VMEM((16, 128), jnp.int32),
            pltpu.VMEM((num_groups,), f32),
            pltpu.VMEM_SHARED((num_groups,), f32),
        ],
    )
    return kern(y2, g2)


def _sc_loss_body(es_hbm, ets_hbm, g_hbm, dtp_hbm, ce_hbm, cnt_hbm,
                  es_v, ets_v, g_v, zero_v, acc_v, dent_loc, dtp_loc,
                  dens_loc, dens_sh):
    cid = lax.axis_index("c")
    sid = lax.axis_index("s")
    row0 = sid * 32

    pltpu.sync_copy(es_hbm.at[pl.ds(row0, 32)], es_v)
    pltpu.sync_copy(ets_hbm.at[pl.ds(row0, 32)], ets_v)
    pltpu.sync_copy(g_hbm.at[pl.ds(row0, 32)], g_v)
    pltpu.sync_copy(dtp_hbm, dtp_loc)

    @pl.when(sid == 0)
    def _zero():
        def _z(i, _):
            zero_v[pl.ds(16 * i, 16)] = jnp.zeros((16,), jnp.float32)
            return 0
        lax.fori_loop(0, 128, _z, 0)
        pltpu.sync_copy(zero_v, dens_sh)

    def _comb(i, _):
        c = pl.ds(16 * i, 16)
        dent_loc[c] = dtp_loc[0, c] + dtp_loc[1, c]
        return 0
    lax.fori_loop(0, 128, _comb, 0)

    plsc.subcore_barrier()

    def _scat(j, _):
        pltpu.sync_copy(es_v.at[j], dens_sh.at[g_v.at[j]], add=True)
        return 0
    lax.fori_loop(0, 32, _scat, 0)

    def _elem(r, _):
        for j in range(8):
            c = pl.ds(16 * j, 16)
            gv = g_v[r, c]
            dtv = plsc.load_gather(dent_loc, [gv])
            acc_v[...] = acc_v[...] + ets_v[r, c] / (dtv + _EPS)
        return 0
    acc_v[...] = jnp.zeros((16,), jnp.float32)

    @pl.when(cid == jnp.where(sid >= 8, 1, 0))
    def _elem_half():
        lax.fori_loop(0, 32, _elem, 0)

    plsc.subcore_barrier()

    w = cid * 16 + sid
    base = w * 64
    pltpu.sync_copy(dens_sh.at[pl.ds(base, 64)], dens_loc)
    ce = jnp.zeros((16,), jnp.float32)
    ct = jnp.zeros((16,), jnp.float32)
    for k in range(4):
        ds = dens_loc[pl.ds(16 * k, 16)]
        dt = dent_loc[pl.ds(base + 16 * k, 16)]
        nonempty = dt > 0.0
        zero16 = jnp.zeros((16,), jnp.float32)
        ce = ce + jnp.where(nonempty, _log_sc(ds + _EPS), zero16)
        ct = ct + jnp.where(nonempty, jnp.ones((16,), jnp.float32), zero16)

    acc_v[...] = ce - acc_v[...]
    pltpu.sync_copy(acc_v, ce_hbm.at[w])
    acc_v[...] = ct
    pltpu.sync_copy(acc_v, cnt_hbm.at[w])


def _run_sc_loss(es2, ets2, g2, dent_part, num_groups):
    mesh = plsc.VectorSubcoreMesh(core_axis_name="c", subcore_axis_name="s")
    f32 = jnp.float32
    kern = pl.kernel(
        _sc_loss_body,
        compiler_params=pltpu.CompilerParams(needs_layout_passes=False),
        out_type=(
            jax.ShapeDtypeStruct((32, 16), f32),
            jax.ShapeDtypeStruct((32, 16), f32),
        ),
        mesh=mesh,
        scratch_types=[
            pltpu.VMEM((32, 128), f32),
            pltpu.VMEM((32, 128), f32),
            pltpu.VMEM((32, 128), jnp.int32),
            pltpu.VMEM((num_groups,), f32),
            pltpu.VMEM((16,), f32),
            pltpu.VMEM((num_groups,), f32),
            pltpu.VMEM((2, num_groups), f32),
            pltpu.VMEM((64,), f32),
            pltpu.VMEM_SHARED((num_groups,), f32),
        ],
    )
    return kern(es2, ets2, g2, dent_part)


def kernel(x, y, g, W1, b1, W2, b2, W3, b3):
    n = x.shape[0]
    num_groups = 2048
    rows = n // 128
    block_m = 16384
    nb = n // block_m

    y2 = y.reshape(rows, 128)
    g2 = g.reshape(rows, 128)
    dent_part = _run_sc_dent(y2, g2, num_groups)
    es, ets = _run_mlp(x, y.reshape(nb, 1, block_m), W1, b1, W2, b2, W3, b3,
                       block_m)
    ce_part, cnt_part = _run_sc_loss(
        es.reshape(rows, 128), ets.reshape(rows, 128), g2, dent_part,
        num_groups)
    return jnp.sum(ce_part) / jnp.maximum(jnp.sum(cnt_part), 1.0)


# speedup vs baseline: 1.0479x; 1.0479x over previous
import functools

import jax
import jax.numpy as jnp
from jax import lax
from jax.experimental import pallas as pl
from jax.experimental.pallas import tpu as pltpu
from jax.experimental.pallas import tpu_sc as plsc

_EPS = 1e-09


def _log_sc(x):
    xi = lax.bitcast_convert_type(x, jnp.int32)
    t = (xi.astype(jnp.float32) - 1064866805.0) * 8.262958405176314e-08
    for _ in range(3):
        t = t + x * jnp.exp(-t) - 1.0
    return t


def _mlp_body(x_ref, y_ref, w1_ref, b1_ref, w2_ref, b2_ref, w3_ref, b3_ref,
              es_ref, ets_ref):
    xh = x_ref[...].astype(jnp.bfloat16)
    w1h = (w1_ref[...] * 0.5).astype(jnp.bfloat16)
    a = jnp.dot(xh, w1h, preferred_element_type=jnp.float32) \
        + 0.5 * b1_ref[...]
    h = a + a * jnp.tanh(a)
    w2h = (w2_ref[...] * 0.5).astype(jnp.bfloat16)
    a = jnp.dot(h.astype(jnp.bfloat16), w2h,
                preferred_element_type=jnp.float32) + 0.5 * b2_ref[...]
    h = a + a * jnp.tanh(a)
    s = lax.dot_general(
        w3_ref[...].astype(jnp.bfloat16), h.astype(jnp.bfloat16),
        dimension_numbers=(((1,), (1,)), ((), ())),
        preferred_element_type=jnp.float32,
    ) + b3_ref[0]
    es_ref[...] = jnp.exp(s)[None]
    ets_ref[...] = (jnp.exp(y_ref[0]) * s)[None]


def _run_mlp(x, y3, W1, b1, W2, b2, W3, b3, block_m):
    n, d = x.shape
    h = W1.shape[1]
    grid = (n // block_m,)
    nb = n // block_m
    vec = pl.BlockSpec((1, 1, block_m), lambda i: (i, 0, 0))
    out3 = jax.ShapeDtypeStruct((nb, 1, block_m), jnp.float32)
    return pl.pallas_call(
        _mlp_body,
        grid=grid,
        in_specs=[
            pl.BlockSpec((block_m, d), lambda i: (i, 0)),
            vec,
            pl.BlockSpec((d, h), lambda i: (0, 0)),
            pl.BlockSpec((1, h), lambda i: (0, 0)),
            pl.BlockSpec((h, h), lambda i: (0, 0)),
            pl.BlockSpec((1, h), lambda i: (0, 0)),
            pl.BlockSpec((1, h), lambda i: (0, 0)),
            pl.BlockSpec(memory_space=pltpu.SMEM),
        ],
        out_specs=(vec, vec),
        out_shape=(out3, out3),
    )(x, y3, W1, b1.reshape(1, h), W2, b2.reshape(1, h), W3.reshape(1, h), b3)


def _sc_dent_body(y_hbm, g_hbm, dtp_hbm,
                  y_v, g_v, zero_v, dent_sh):
    cid = lax.axis_index("c")
    sid = lax.axis_index("s")
    row0 = cid * 256 + sid * 16

    pltpu.sync_copy(y_hbm.at[pl.ds(row0, 16)], y_v)
    pltpu.sync_copy(g_hbm.at[pl.ds(row0, 16)], g_v)

    def _expy_row(r, _):
        for j in range(8):
            c = pl.ds(16 * j, 16)
            y_v[r, c] = jnp.exp(y_v[r, c])
        return 0
    lax.fori_loop(0, 16, _expy_row, 0)

    @pl.when(sid == 0)
    def _zero():
        def _z(i, _):
            zero_v[pl.ds(16 * i, 16)] = jnp.zeros((16,), jnp.float32)
            return 0
        lax.fori_loop(0, 128, _z, 0)
        pltpu.sync_copy(zero_v, dent_sh)

    plsc.subcore_barrier()

    def _scat(j, _):
        pltpu.sync_copy(y_v.at[j], dent_sh.at[g_v.at[j]], add=True)
        return 0
    lax.fori_loop(0, 16, _scat, 0)

    plsc.subcore_barrier()

    @pl.when(sid == 0)
    def _out():
        pltpu.sync_copy(dent_sh, dtp_hbm.at[cid])


def _run_sc_dent(y2, g2, num_groups):
    mesh = plsc.VectorSubcoreMesh(core_axis_name="c", subcore_axis_name="s")
    f32 = jnp.float32
    kern = pl.kernel(
        _sc_dent_body,
        compiler_params=pltpu.CompilerParams(needs_layout_passes=False),
        out_type=jax.ShapeDtypeStruct((2, num_groups), f32),
        mesh=mesh,
        scratch_types=[
            pltpu.VMEM((16, 128), f32),
            pltpu.VMEM((16, 128), jnp.int32),
            pltpu.VMEM((num_groups,), f32),
            pltpu.VMEM_SHARED((num_groups,), f32),
        ],
    )
    return kern(y2, g2)


def _sc_loss_body(es_hbm, ets_hbm, g_hbm, dtp_hbm, ce_hbm, cnt_hbm,
                  es_v, ets_v, g_v, zero_v, acc_v, dent_loc, dtp_loc,
                  dens_loc, dens_sh):
    cid = lax.axis_index("c")
    sid = lax.axis_index("s")
    row0 = sid * 32

    pltpu.sync_copy(es_hbm.at[pl.ds(row0, 32)], es_v)
    pltpu.sync_copy(ets_hbm.at[pl.ds(row0, 32)], ets_v)
    pltpu.sync_copy(g_hbm.at[pl.ds(row0, 32)], g_v)
    pltpu.sync_copy(dtp_hbm, dtp_loc)

    @pl.when(sid == 0)
    def _zero():
        def _z(i, _):
            zero_v[pl.ds(16 * i, 16)] = jnp.zeros((16,), jnp.float32)
            return 0
        lax.fori_loop(0, 128, _z, 0)
        pltpu.sync_copy(zero_v, dens_sh)

    def _comb(i, _):
        c = pl.ds(16 * i, 16)
        dent_loc[c] = dtp_loc[0, c] + dtp_loc[1, c]
        return 0
    lax.fori_loop(0, 128, _comb, 0)

    plsc.subcore_barrier()

    def _scat(j, _):
        pltpu.sync_copy(es_v.at[j], dens_sh.at[g_v.at[j]], add=True)
        return 0
    lax.fori_loop(0, 32, _scat, 0)

    def _elem(r, _):
        for j in range(8):
            c = pl.ds(16 * j, 16)
            gv = g_v[r, c]
            dtv = plsc.load_gather(dent_loc, [gv])
            acc_v[...] = acc_v[...] + ets_v[r, c] / (dtv + _EPS)
        return 0
    acc_v[...] = jnp.zeros((16,), jnp.float32)

    @pl.when(cid == jnp.where(sid >= 8, 1, 0))
    def _elem_half():
        lax.fori_loop(0, 32, _elem, 0)

    plsc.subcore_barrier()

    w = cid * 16 + sid
    base = w * 64
    pltpu.sync_copy(dens_sh.at[pl.ds(base, 64)], dens_loc)
    ce = jnp.zeros((16,), jnp.float32)
    ct = jnp.zeros((16,), jnp.float32)
    for k in range(4):
        ds = dens_loc[pl.ds(16 * k, 16)]
        dt = dent_loc[pl.ds(base + 16 * k, 16)]
        nonempty = dt > 0.0
        zero16 = jnp.zeros((16,), jnp.float32)
        ce = ce + jnp.where(nonempty, _log_sc(ds + _EPS), zero16)
        ct = ct + jnp.where(nonempty, jnp.ones((16,), jnp.float32), zero16)

    acc_v[...] = ce - acc_v[...]
    pltpu.sync_copy(acc_v, ce_hbm.at[w])
    acc_v[...] = ct
    pltpu.sync_copy(acc_v, cnt_hbm.at[w])


def _run_sc_loss(es2, ets2, g2, dent_part, num_groups):
    mesh = plsc.VectorSubcoreMesh(core_axis_name="c", subcore_axis_name="s")
    f32 = jnp.float32
    kern = pl.kernel(
        _sc_loss_body,
        compiler_params=pltpu.CompilerParams(needs_layout_passes=False),
        out_type=(
            jax.ShapeDtypeStruct((32, 16), f32),
            jax.ShapeDtypeStruct((32, 16), f32),
        ),
        mesh=mesh,
        scratch_types=[
            pltpu.VMEM((32, 128), f32),
            pltpu.VMEM((32, 128), f32),
            pltpu.VMEM((32, 128), jnp.int32),
            pltpu.VMEM((num_groups,), f32),
            pltpu.VMEM((16,), f32),
            pltpu.VMEM((num_groups,), f32),
            pltpu.VMEM((2, num_groups), f32),
            pltpu.VMEM((64,), f32),
            pltpu.VMEM_SHARED((num_groups,), f32),
        ],
    )
    return kern(es2, ets2, g2, dent_part)


def kernel(x, y, g, W1, b1, W2, b2, W3, b3):
    n = x.shape[0]
    num_groups = 2048
    rows = n // 128
    block_m = 8192
    nb = n // block_m

    y2 = y.reshape(rows, 128)
    g2 = g.reshape(rows, 128)
    dent_part = _run_sc_dent(y2, g2, num_groups)
    es, ets = _run_mlp(x, y.reshape(nb, 1, block_m), W1, b1, W2, b2, W3, b3,
                       block_m)
    ce_part, cnt_part = _run_sc_loss(
        es.reshape(rows, 128), ets.reshape(rows, 128), g2, dent_part,
        num_groups)
    return jnp.sum(ce_part) / jnp.maximum(jnp.sum(cnt_part), 1.0)
